# Initial kernel scaffold; baseline (speedup 1.0000x reference)
import jax
import jax.numpy as jnp
from jax.experimental import pallas as pl


def kernel(update_step, requests, latency, active_mask, params, node_type, edge_index):
    raise NotImplementedError("write your pallas kernel here")


# speedup vs baseline: 227.9893x; 227.9893x over previous
import functools

import numpy as np
import jax
import jax.numpy as jnp
from jax import lax
from jax.experimental import pallas as pl
from jax.experimental.pallas import tpu as pltpu
from jax.experimental.pallas import tpu_sc as plsc

_L = 15
_HEADS = 4
_OUT_H = 3
_HID = 12
_NEG = -3.0e38
_RB = 2048
_EB = 256
_NC = 2
_NS = 16

_HEADMAP = [0, 0, 0, 1, 1, 1, 2, 2, 2, 3, 3, 3, 0, 1, 2, 3]

_R_NP = np.zeros((_HEADS, _HID), np.float32)
for _h in range(_HEADS):
    for _k in range(_OUT_H):
        _R_NP[_h, _h * _OUT_H + _k] = 1.0


def _leaky(x):
    return jnp.where(x >= 0, x, 0.2 * x)


def _svb16(s4, r):
    return jnp.concatenate(
        [jnp.dot(s4, r, preferred_element_type=jnp.float32), s4], axis=1)


def _stats_body(req_ref, mean_ref, std_ref, *, n):
    npad = req_ref.shape[0]
    rid = lax.broadcasted_iota(jnp.int32, (npad, 1), 0)
    req = req_ref[...]
    vst = jnp.logical_and(rid >= _L, rid < n).astype(jnp.float32)
    cnt = float(n - _L)
    mean = jnp.sum(req * vst) / cnt
    dev = (req - mean) * vst
    std = jnp.sqrt(jnp.sum(dev * dev) / (cnt - 1.0))
    mean_ref[0, 0] = mean
    std_ref[0, 0] = std


def _call_stats(req, npad, n):
    return pl.pallas_call(
        functools.partial(_stats_body, n=n),
        out_specs=(
            pl.BlockSpec(memory_space=pltpu.SMEM),
            pl.BlockSpec(memory_space=pltpu.SMEM),
        ),
        out_shape=(
            jax.ShapeDtypeStruct((1, 1), jnp.float32),
            jax.ShapeDtypeStruct((1, 1), jnp.float32),
        ),
    )(req)


def _pre0_body(nt_ref, req_ref, upd_ref, mean_ref, std_ref, emb_ref, wa_ref,
               wb_ref, wc_ref, r_ref,
               t1a_ref, t1b_ref, t2b_ref, sv_ref, ms_ref, md_ref,
               *, n, nblk):
    i = pl.program_id(0)
    rows = nt_ref.shape[0]
    rid = i * _RB + lax.broadcasted_iota(jnp.int32, (rows, 1), 0)
    req = req_ref[...]
    mean = mean_ref[0, 0]
    std = std_ref[0, 0]
    reqf = jnp.where(rid < _L, req, (req - mean) / std)
    nt = nt_ref[...]
    emb = emb_ref[...]
    xe = jnp.zeros((rows, 3), jnp.float32)
    for t in range(4):
        xe = jnp.where(nt == t, emb[t:t + 1, :], xe)
    x0 = jnp.concatenate(
        [xe, reqf, upd_ref[...], jnp.zeros((rows, 3), jnp.float32)], axis=1)
    t1a = jnp.dot(x0, wa_ref[...], preferred_element_type=jnp.float32)
    h = jnp.dot(x0, wb_ref[...], preferred_element_type=jnp.float32)
    t2b = jnp.dot(x0, wc_ref[...], preferred_element_type=jnp.float32)
    t1a_ref[...] = t1a
    t1b_ref[...] = jnp.concatenate([h, jnp.ones((rows, 4), jnp.float32)],
                                   axis=1)
    t2b_ref[...] = t2b
    valid = rid < n
    bms = jnp.max(jnp.where(valid, t1a[:, 12:16], _NEG), axis=0,
                  keepdims=True)
    bmd = jnp.max(jnp.where(valid, t2b[:, 12:16], _NEG), axis=0,
                  keepdims=True)

    @pl.when(i == 0)
    def _():
        ms_ref[...] = bms
        md_ref[...] = bmd

    @pl.when(i > 0)
    def _():
        ms_ref[...] = jnp.maximum(ms_ref[...], bms)
        md_ref[...] = jnp.maximum(md_ref[...], bmd)

    @pl.when(i == nblk - 1)
    def _():
        sv_ref[...] = _svb16(_leaky(ms_ref[...] + md_ref[...]), r_ref[...])


def _call_pre0(nt, req, upd, emb, wa, wb, wc, r, npad, n):
    mean, std = _call_stats(req, npad, n)
    nblk = npad // _RB
    mat = lambda i: (i, 0)
    fix = lambda i: (0, 0)
    d_in = wa.shape[0]
    return pl.pallas_call(
        functools.partial(_pre0_body, n=n, nblk=nblk),
        grid=(nblk,),
        in_specs=[
            pl.BlockSpec((_RB, 1), mat),
            pl.BlockSpec((_RB, 1), mat),
            pl.BlockSpec((_RB, 1), mat),
            pl.BlockSpec(memory_space=pltpu.SMEM),
            pl.BlockSpec(memory_space=pltpu.SMEM),
            pl.BlockSpec((4, 3), fix),
            pl.BlockSpec((d_in, 16), fix),
            pl.BlockSpec((d_in, 12), fix),
            pl.BlockSpec((d_in, 16), fix),
            pl.BlockSpec((4, 12), fix),
        ],
        out_specs=(
            pl.BlockSpec((_RB, 16), mat),
            pl.BlockSpec((_RB, 16), mat),
            pl.BlockSpec((_RB, 16), mat),
            pl.BlockSpec((1, 16), fix),
        ),
        out_shape=(
            jax.ShapeDtypeStruct((npad, 16), jnp.float32),
            jax.ShapeDtypeStruct((npad, 16), jnp.float32),
            jax.ShapeDtypeStruct((npad, 16), jnp.float32),
            jax.ShapeDtypeStruct((1, 16), jnp.float32),
        ),
        scratch_shapes=[
            pltpu.VMEM((1, 4), jnp.float32),
            pltpu.VMEM((1, 4), jnp.float32),
        ],
    )(nt, req, upd, mean, std, emb, wa, wb, wc, r)


def _mid_epilogue(a_ref, t1a_ref, t1b_ref, t2b_ref, sv_ref, b_ref, r_ref):
    asum = a_ref[0] + a_ref[1]
    h = t1b_ref[:, :12]
    r = r_ref[...]
    wself = jnp.exp(
        _leaky(t1a_ref[:, 12:16] + t2b_ref[:, 12:16]) - sv_ref[0:1, 12:16])
    num = asum[:, :12] + h * jnp.dot(wself, r,
                                     preferred_element_type=jnp.float32)
    den4 = asum[:, 12:16] + wself
    den = jnp.dot(den4, r, preferred_element_type=jnp.float32)
    return num / (den + 1e-16) + b_ref[0:1, :12]


def _mid_body(a_ref, t1a_ref, t1b_ref, t2b_ref, sv_ref, b_ref, r_ref,
              wa_ref, wb_ref, wc_ref,
              t1ao_ref, t1bo_ref, t2bo_ref, svo_ref, ms_ref, md_ref,
              *, n, nblk):
    i = pl.program_id(0)
    out = _mid_epilogue(a_ref, t1a_ref, t1b_ref, t2b_ref, sv_ref, b_ref,
                        r_ref)
    x = jnp.maximum(out, 0.0)
    t1a = jnp.dot(x, wa_ref[...], preferred_element_type=jnp.float32)
    h = jnp.dot(x, wb_ref[...], preferred_element_type=jnp.float32)
    t2b = jnp.dot(x, wc_ref[...], preferred_element_type=jnp.float32)
    t1ao_ref[...] = t1a
    t1bo_ref[...] = jnp.concatenate(
        [h, jnp.ones((h.shape[0], 4), jnp.float32)], axis=1)
    t2bo_ref[...] = t2b
    rid = i * _RB + lax.broadcasted_iota(jnp.int32, (t1a.shape[0], 1), 0)
    valid = rid < n
    bms = jnp.max(jnp.where(valid, t1a[:, 12:16], _NEG), axis=0,
                  keepdims=True)
    bmd = jnp.max(jnp.where(valid, t2b[:, 12:16], _NEG), axis=0,
                  keepdims=True)

    @pl.when(i == 0)
    def _():
        ms_ref[...] = bms
        md_ref[...] = bmd

    @pl.when(i > 0)
    def _():
        ms_ref[...] = jnp.maximum(ms_ref[...], bms)
        md_ref[...] = jnp.maximum(md_ref[...], bmd)

    @pl.when(i == nblk - 1)
    def _():
        svo_ref[...] = _svb16(_leaky(ms_ref[...] + md_ref[...]), r_ref[...])


def _call_mid(a, t1a, t1b, t2b, sv, b16, r, wa, wb, wc, npad, n):
    nblk = npad // _RB
    mat = lambda i: (i, 0)
    fix = lambda i: (0, 0)
    return pl.pallas_call(
        functools.partial(_mid_body, n=n, nblk=nblk),
        grid=(nblk,),
        in_specs=[
            pl.BlockSpec((2, _RB, 16), lambda i: (0, i, 0)),
            pl.BlockSpec((_RB, 16), mat),
            pl.BlockSpec((_RB, 16), mat),
            pl.BlockSpec((_RB, 16), mat),
            pl.BlockSpec((1, 16), fix),
            pl.BlockSpec((1, 16), fix),
            pl.BlockSpec((4, 12), fix),
            pl.BlockSpec((12, 16), fix),
            pl.BlockSpec((12, 12), fix),
            pl.BlockSpec((12, 16), fix),
        ],
        out_specs=(
            pl.BlockSpec((_RB, 16), mat),
            pl.BlockSpec((_RB, 16), mat),
            pl.BlockSpec((_RB, 16), mat),
            pl.BlockSpec((1, 16), fix),
        ),
        out_shape=(
            jax.ShapeDtypeStruct((npad, 16), jnp.float32),
            jax.ShapeDtypeStruct((npad, 16), jnp.float32),
            jax.ShapeDtypeStruct((npad, 16), jnp.float32),
            jax.ShapeDtypeStruct((1, 16), jnp.float32),
        ),
        scratch_shapes=[
            pltpu.VMEM((1, 4), jnp.float32),
            pltpu.VMEM((1, 4), jnp.float32),
        ],
    )(a, t1a, t1b, t2b, sv, b16, r, wa, wb, wc)


def _last_body(a_ref, t1a_ref, t1b_ref, t2b_ref, sv_ref, b_ref, r_ref,
               hl_ref):
    hl_ref[...] = _mid_epilogue(a_ref, t1a_ref, t1b_ref, t2b_ref, sv_ref,
                                b_ref, r_ref)


def _call_last(a, t1a, t1b, t2b, sv, b16, r, npad):
    nblk = npad // _RB
    mat = lambda i: (i, 0)
    fix = lambda i: (0, 0)
    return pl.pallas_call(
        _last_body,
        grid=(nblk,),
        in_specs=[
            pl.BlockSpec((2, _RB, 16), lambda i: (0, i, 0)),
            pl.BlockSpec((_RB, 16), mat),
            pl.BlockSpec((_RB, 16), mat),
            pl.BlockSpec((_RB, 16), mat),
            pl.BlockSpec((1, 16), fix),
            pl.BlockSpec((1, 16), fix),
            pl.BlockSpec((4, 12), fix),
        ],
        out_specs=pl.BlockSpec((_RB, 12), mat),
        out_shape=jax.ShapeDtypeStruct((npad, 12), jnp.float32),
    )(a, t1a, t1b, t2b, sv, b16, r)


def _sample_update(i, zl, rl, rid, m_ref, s_ref, bv_ref, bi_ref):
    bmax_rl = jnp.max(rl)
    bmax_z = jnp.max(zl)
    bidx = jnp.min(jnp.where(zl == bmax_z, rid, jnp.int32(2**30)))

    @pl.when(i == 0)
    def _():
        m_ref[0, 0] = _NEG
        s_ref[0, 0] = 0.0
        bv_ref[0, 0] = _NEG
        bi_ref[0, 0] = 0

    m_old = m_ref[0, 0]
    m_new = jnp.maximum(m_old, bmax_rl)
    s_ref[0, 0] = s_ref[0, 0] * jnp.exp(m_old - m_new) + jnp.sum(
        jnp.exp(rl - m_new))
    m_ref[0, 0] = m_new
    upd = bmax_z > bv_ref[0, 0]
    bv_ref[0, 0] = jnp.where(upd, bmax_z, bv_ref[0, 0])
    bi_ref[0, 0] = jnp.where(upd, bidx, bi_ref[0, 0])


def _fc_body(h_ref, am_ref, g_ref, w1_ref, b1_ref, wa_ref, ba_ref, wb_ref,
             bb_ref, wc_ref, bc_ref, wo_ref, bo_ref,
             rl_ref, act_ref, lse_ref, m_ref, s_ref, bv_ref, bi_ref,
             *, n, nblk):
    i = pl.program_id(0)
    x = jnp.maximum(
        jnp.dot(h_ref[...], w1_ref[...], preferred_element_type=jnp.float32)
        + b1_ref[...], 0.0)
    for w, b in ((wa_ref, ba_ref), (wb_ref, bb_ref), (wc_ref, bc_ref)):
        x = jnp.maximum(
            jnp.dot(x, w[...], preferred_element_type=jnp.float32) + b[...],
            0.0)
    sc = jnp.dot(x, wo_ref[...], preferred_element_type=jnp.float32) \
        + bo_ref[...]
    am = am_ref[...]
    rid = i * _RB + lax.broadcasted_iota(jnp.int32, (sc.shape[0], 1), 0)
    rm_head = jnp.where(am == 0.0, -jnp.inf,
                        jnp.where(jnp.isneginf(am), 0.0, am))
    rm = jnp.where(rid < _L, rm_head, am)
    rl = jnp.where(rid < n, sc + rm, _NEG)
    rl_ref[...] = rl
    zl = jnp.where(rid < n, rl + g_ref[...], _NEG)
    _sample_update(i, zl, rl, rid, m_ref, s_ref, bv_ref, bi_ref)

    @pl.when(i == nblk - 1)
    def _():
        lse_ref[0, 0] = m_ref[0, 0] + jnp.log(s_ref[0, 0])
        act_ref[0, 0] = bi_ref[0, 0]


def _call_fc(hl1, am, g1, p, npad, n):
    nblk = npad // _RB
    mat = lambda i: (i, 0)
    fix = lambda i: (0, 0)
    return pl.pallas_call(
        functools.partial(_fc_body, n=n, nblk=nblk),
        grid=(nblk,),
        in_specs=[
            pl.BlockSpec((_RB, 12), mat),
            pl.BlockSpec((_RB, 1), mat),
            pl.BlockSpec((_RB, 1), mat),
            pl.BlockSpec((12, 128), fix),
            pl.BlockSpec((1, 128), fix),
            pl.BlockSpec((128, 128), fix),
            pl.BlockSpec((1, 128), fix),
            pl.BlockSpec((128, 128), fix),
            pl.BlockSpec((1, 128), fix),
            pl.BlockSpec((128, 128), fix),
            pl.BlockSpec((1, 128), fix),
            pl.BlockSpec((128, 1), fix),
            pl.BlockSpec((1, 1), fix),
        ],
        out_specs=(
            pl.BlockSpec((_RB, 1), mat),
            pl.BlockSpec((1, 1), fix, memory_space=pltpu.SMEM),
            pl.BlockSpec((1, 1), fix, memory_space=pltpu.SMEM),
        ),
        out_shape=(
            jax.ShapeDtypeStruct((npad, 1), jnp.float32),
            jax.ShapeDtypeStruct((1, 1), jnp.int32),
            jax.ShapeDtypeStruct((1, 1), jnp.float32),
        ),
        scratch_shapes=[
            pltpu.SMEM((1, 1), jnp.float32),
            pltpu.SMEM((1, 1), jnp.float32),
            pltpu.SMEM((1, 1), jnp.float32),
            pltpu.SMEM((1, 1), jnp.int32),
        ],
    )(hl1, am, g1, p['fc1_W'], p['fc1_b'][None, :],
      p['mlp'][0]['W'], p['mlp'][0]['b'][None, :],
      p['mlp'][1]['W'], p['mlp'][1]['b'][None, :],
      p['mlp'][2]['W'], p['mlp'][2]['b'][None, :],
      p['out_W'], p['out_b'][None, :])


def _k2_body(h_ref, am_ref, g_ref, p_ref, a1_ref,
             nl_ref, act_ref, lse_ref, m_ref, s_ref, bv_ref, bi_ref,
             *, n, nblk):
    i = pl.program_id(0)
    v = jnp.dot(h_ref[...], p_ref[...], preferred_element_type=jnp.float32)
    rid = i * _RB + lax.broadcasted_iota(jnp.int32, (v.shape[0], 1), 0)
    a1 = a1_ref[0, 0]
    mask2 = jnp.where(rid == a1, 0.0, am_ref[...])
    nl = jnp.where(rid < n, v + mask2, _NEG)
    nl_ref[...] = nl
    zl = jnp.where(rid < n, nl + g_ref[...], _NEG)
    _sample_update(i, zl, nl, rid, m_ref, s_ref, bv_ref, bi_ref)

    @pl.when(i == nblk - 1)
    def _():
        lse_ref[0, 0] = m_ref[0, 0] + jnp.log(s_ref[0, 0])
        act_ref[0, 0] = bi_ref[0, 0]


def _call_k2(hl1, am, g2, pvec, a1, npad, n):
    nblk = npad // _RB
    mat = lambda i: (i, 0)
    fix = lambda i: (0, 0)
    return pl.pallas_call(
        functools.partial(_k2_body, n=n, nblk=nblk),
        grid=(nblk,),
        in_specs=[
            pl.BlockSpec((_RB, 12), mat),
            pl.BlockSpec((_RB, 1), mat),
            pl.BlockSpec((_RB, 1), mat),
            pl.BlockSpec((12, 1), fix),
            pl.BlockSpec((1, 1), fix, memory_space=pltpu.SMEM),
        ],
        out_specs=(
            pl.BlockSpec((_RB, 1), mat),
            pl.BlockSpec((1, 1), fix, memory_space=pltpu.SMEM),
            pl.BlockSpec((1, 1), fix, memory_space=pltpu.SMEM),
        ),
        out_shape=(
            jax.ShapeDtypeStruct((npad, 1), jnp.float32),
            jax.ShapeDtypeStruct((1, 1), jnp.int32),
            jax.ShapeDtypeStruct((1, 1), jnp.float32),
        ),
        scratch_shapes=[
            pltpu.SMEM((1, 1), jnp.float32),
            pltpu.SMEM((1, 1), jnp.float32),
            pltpu.SMEM((1, 1), jnp.float32),
            pltpu.SMEM((1, 1), jnp.int32),
        ],
    )(hl1, am, g2, pvec, a1)


def _build_sc(npad, ep):
    pt = ep // (_NC * _NS)
    nb = pt // _EB
    rows_tile = npad // _NS
    zch = rows_tile // 16
    mesh = plsc.VectorSubcoreMesh(core_axis_name="c", subcore_axis_name="s",
                                  num_cores=_NC)

    @functools.partial(
        pl.kernel,
        out_type=jax.ShapeDtypeStruct((_NC, npad, 16), jnp.float32),
        mesh=mesh,
        compiler_params=pltpu.CompilerParams(use_tc_tiling_on_sc=False),
        scratch_types=[
            pltpu.VMEM_SHARED((npad, 16), jnp.float32),
            pltpu.VMEM((16,), jnp.float32),
            pltpu.VMEM((_EB,), jnp.int32),
            pltpu.VMEM((_EB,), jnp.int32),
            pltpu.VMEM((_EB, 16), jnp.float32),
            pltpu.VMEM((_EB, 16), jnp.float32),
            pltpu.VMEM((_EB, 16), jnp.float32),
            pltpu.VMEM((_EB, 16), jnp.float32),
            pltpu.SemaphoreType.DMA,
            pltpu.SemaphoreType.DMA,
            pltpu.SemaphoreType.DMA,
        ],
    )
    def sc_pass(src_h, dst_h, t1a_h, t1b_h, t2b_h, sv_h, out_h,
                a_sh, svv, src_v, dst_v, ba_v, bb_v, bc_v, out_v,
                sem1, sem2, sem3):
        cid = lax.axis_index("c")
        sid = lax.axis_index("s")
        zero16 = jnp.zeros((16,), jnp.float32)
        base_rows = sid * rows_tile

        @pl.loop(0, _EB)
        def _(i):
            out_v[i, :] = zero16

        @pl.loop(0, 16)
        def _(k):
            pltpu.sync_copy(out_v.at[pl.ds(0, zch), :],
                            a_sh.at[pl.ds(base_rows + k * zch, zch), :])

        plsc.subcore_barrier()
        pltpu.sync_copy(sv_h, svv)
        svb = svv[...]
        tile_base = (cid * _NS + sid) * pt

        @pl.loop(0, nb)
        def _(b):
            base = tile_base + b * _EB
            pltpu.sync_copy(src_h.at[pl.ds(base, _EB)], src_v)
            pltpu.sync_copy(dst_h.at[pl.ds(base, _EB)], dst_v)
            d1 = pltpu.async_copy(t1a_h.at[src_v], ba_v, sem1)
            d2 = pltpu.async_copy(t1b_h.at[src_v], bb_v, sem2)
            d3 = pltpu.async_copy(t2b_h.at[dst_v], bc_v, sem3)
            d1.wait()
            d2.wait()
            d3.wait()
            for j in range(_EB):
                al = ba_v[j, :] + bc_v[j, :]
                al = jnp.where(al >= 0.0, al, al * 0.2)
                w = jnp.exp(al - svb)
                out_v[j, :] = bb_v[j, :] * w
            pltpu.sync_copy(out_v, a_sh.at[dst_v], add=True)

        plsc.subcore_barrier()

        @pl.loop(0, 16)
        def _(k):
            sl = pl.ds(base_rows + k * zch, zch)
            pltpu.sync_copy(a_sh.at[sl, :], out_v.at[pl.ds(0, zch), :])
            pltpu.sync_copy(out_v.at[pl.ds(0, zch), :], out_h.at[cid, sl, :])

    return sc_pass


def _fold_gat(gat, r, d_pad):
    w = gat['W'].astype(jnp.float32)
    d_in = w.shape[0]
    ssrc = r.T * gat['att_src'].reshape(_HID)[:, None]
    sdst = r.T * gat['att_dst'].reshape(_HID)[:, None]
    hm = jnp.asarray(_HEADMAP)
    wa = (w @ ssrc)[:, hm]
    wc = (w @ sdst)[:, hm]
    wb = w
    if d_pad > d_in:
        pad = ((0, d_pad - d_in), (0, 0))
        wa = jnp.pad(wa, pad)
        wb = jnp.pad(wb, pad)
        wc = jnp.pad(wc, pad)
    return wa, wb, wc


def kernel(update_step, requests, latency, active_mask, params, node_type,
           edge_index):
    del latency
    n = node_type.shape[0]
    e = edge_index.shape[1]
    npad = -(-n // _RB) * _RB
    epb = _NC * _NS * _EB
    ep = -(-e // epb) * epb
    f32 = jnp.float32

    def col(a, dtype=f32):
        return jnp.pad(a.astype(dtype), (0, npad - n))[:, None]

    nt = col(node_type, jnp.int32)
    req = col(requests)
    upd = col(update_step)
    am = col(active_mask)
    src = jnp.concatenate([edge_index[0].astype(jnp.int32),
                           jnp.full((ep - e,), n, jnp.int32)])
    dst = jnp.concatenate([edge_index[1].astype(jnp.int32),
                           jnp.full((ep - e,), n, jnp.int32)])

    r = jnp.asarray(_R_NP)
    gats = params['gats']
    wa0, wb0, wc0 = _fold_gat(gats[0], r, 8)
    t1a, t1b, t2b, sv = _call_pre0(nt, req, upd,
                                   params['type_emb'].astype(f32),
                                   wa0, wb0, wc0, r, npad, n)
    sc_pass = _build_sc(npad, ep)
    for l in range(4):
        b16 = jnp.pad(gats[l]['b'].astype(f32), (0, 4))[None, :]
        a = sc_pass(src, dst, t1a, t1b, t2b, sv.reshape(16))
        if l < 3:
            wa, wb, wc = _fold_gat(gats[l + 1], r, 12)
            t1a, t1b, t2b, sv = _call_mid(a, t1a, t1b, t2b, sv, b16, r,
                                          wa, wb, wc, npad, n)
        else:
            hl1 = _call_last(a, t1a, t1b, t2b, sv, b16, r, npad)

    k1, k2 = jax.random.split(jax.random.key(42))
    g1 = jnp.pad(jax.random.gumbel(k1, (n,), f32), (0, npad - n))[:, None]
    g2 = jnp.pad(jax.random.gumbel(k2, (n,), f32), (0, npad - n))[:, None]
    rl, a1, lse1 = _call_fc(hl1, am, g1, params, npad, n)
    a1s = a1[0, 0]
    row = lax.dynamic_index_in_dim(hl1, a1s, axis=0, keepdims=False)
    pvec = jnp.tanh(row @ params['proj_W'].astype(f32)
                    + params['proj_b'].astype(f32))
    nl, a2, lse2 = _call_k2(hl1, am, g2, pvec[:, None], a1, npad, n)

    rl1 = rl[:n, 0]
    nl1 = nl[:n, 0]
    logits = jnp.stack([rl1, nl1])
    actions = jnp.stack([a1s, a2[0, 0]])
    log_probs = jnp.stack([rl1[a1s] - lse1[0, 0], nl1[a2[0, 0]] - lse2[0, 0]])
    return logits, actions, log_probs


# speedup vs baseline: 241.4925x; 1.0592x over previous
import functools

import numpy as np
import jax
import jax.numpy as jnp
from jax import lax
from jax.experimental import pallas as pl
from jax.experimental.pallas import tpu as pltpu
from jax.experimental.pallas import tpu_sc as plsc

_L = 15
_HEADS = 4
_OUT_H = 3
_HID = 12
_NEG = -3.0e38
_RB = 2048
_EB = 128
_NC = 2
_NS = 16

_HEADMAP = [0, 0, 0, 1, 1, 1, 2, 2, 2, 3, 3, 3, 0, 1, 2, 3]

_R_NP = np.zeros((_HEADS, _HID), np.float32)
for _h in range(_HEADS):
    for _k in range(_OUT_H):
        _R_NP[_h, _h * _OUT_H + _k] = 1.0


def _leaky(x):
    return jnp.where(x >= 0, x, 0.2 * x)


def _svb16(s4, r):
    return jnp.concatenate(
        [jnp.dot(s4, r, preferred_element_type=jnp.float32), s4], axis=1)


def _stats_body(req_ref, mean_ref, std_ref, *, n):
    npad = req_ref.shape[0]
    rid = lax.broadcasted_iota(jnp.int32, (npad, 1), 0)
    req = req_ref[...]
    vst = jnp.logical_and(rid >= _L, rid < n).astype(jnp.float32)
    cnt = float(n - _L)
    mean = jnp.sum(req * vst) / cnt
    dev = (req - mean) * vst
    std = jnp.sqrt(jnp.sum(dev * dev) / (cnt - 1.0))
    mean_ref[0, 0] = mean
    std_ref[0, 0] = std


def _call_stats(req, npad, n):
    return pl.pallas_call(
        functools.partial(_stats_body, n=n),
        out_specs=(
            pl.BlockSpec(memory_space=pltpu.SMEM),
            pl.BlockSpec(memory_space=pltpu.SMEM),
        ),
        out_shape=(
            jax.ShapeDtypeStruct((1, 1), jnp.float32),
            jax.ShapeDtypeStruct((1, 1), jnp.float32),
        ),
    )(req)


def _pre0_body(nt_ref, req_ref, upd_ref, mean_ref, std_ref, emb_ref, wa_ref,
               wb_ref, wc_ref, r_ref,
               t1a_ref, t1b_ref, t2b_ref, sv_ref, ms_ref, md_ref,
               *, n, nblk):
    i = pl.program_id(0)
    rows = nt_ref.shape[0]
    rid = i * _RB + lax.broadcasted_iota(jnp.int32, (rows, 1), 0)
    req = req_ref[...]
    mean = mean_ref[0, 0]
    std = std_ref[0, 0]
    reqf = jnp.where(rid < _L, req, (req - mean) / std)
    nt = nt_ref[...]
    emb = emb_ref[...]
    xe = jnp.zeros((rows, 3), jnp.float32)
    for t in range(4):
        xe = jnp.where(nt == t, emb[t:t + 1, :], xe)
    x0 = jnp.concatenate(
        [xe, reqf, upd_ref[...], jnp.zeros((rows, 3), jnp.float32)], axis=1)
    t1a = jnp.dot(x0, wa_ref[...], preferred_element_type=jnp.float32)
    h = jnp.dot(x0, wb_ref[...], preferred_element_type=jnp.float32)
    t2b = jnp.dot(x0, wc_ref[...], preferred_element_type=jnp.float32)
    t1a_ref[...] = t1a
    t1b_ref[...] = jnp.concatenate([h, jnp.ones((rows, 4), jnp.float32)],
                                   axis=1)
    t2b_ref[...] = t2b
    valid = rid < n
    bms = jnp.max(jnp.where(valid, t1a[:, 12:16], _NEG), axis=0,
                  keepdims=True)
    bmd = jnp.max(jnp.where(valid, t2b[:, 12:16], _NEG), axis=0,
                  keepdims=True)

    @pl.when(i == 0)
    def _():
        ms_ref[...] = bms
        md_ref[...] = bmd

    @pl.when(i > 0)
    def _():
        ms_ref[...] = jnp.maximum(ms_ref[...], bms)
        md_ref[...] = jnp.maximum(md_ref[...], bmd)

    @pl.when(i == nblk - 1)
    def _():
        sv_ref[...] = _svb16(_leaky(ms_ref[...] + md_ref[...]), r_ref[...])


def _call_pre0(nt, req, upd, emb, wa, wb, wc, r, npad, n):
    mean, std = _call_stats(req, npad, n)
    nblk = npad // _RB
    mat = lambda i: (i, 0)
    fix = lambda i: (0, 0)
    d_in = wa.shape[0]
    return pl.pallas_call(
        functools.partial(_pre0_body, n=n, nblk=nblk),
        grid=(nblk,),
        in_specs=[
            pl.BlockSpec((_RB, 1), mat),
            pl.BlockSpec((_RB, 1), mat),
            pl.BlockSpec((_RB, 1), mat),
            pl.BlockSpec(memory_space=pltpu.SMEM),
            pl.BlockSpec(memory_space=pltpu.SMEM),
            pl.BlockSpec((4, 3), fix),
            pl.BlockSpec((d_in, 16), fix),
            pl.BlockSpec((d_in, 12), fix),
            pl.BlockSpec((d_in, 16), fix),
            pl.BlockSpec((4, 12), fix),
        ],
        out_specs=(
            pl.BlockSpec((_RB, 16), mat),
            pl.BlockSpec((_RB, 16), mat),
            pl.BlockSpec((_RB, 16), mat),
            pl.BlockSpec((1, 16), fix),
        ),
        out_shape=(
            jax.ShapeDtypeStruct((npad, 16), jnp.float32),
            jax.ShapeDtypeStruct((npad, 16), jnp.float32),
            jax.ShapeDtypeStruct((npad, 16), jnp.float32),
            jax.ShapeDtypeStruct((1, 16), jnp.float32),
        ),
        scratch_shapes=[
            pltpu.VMEM((1, 4), jnp.float32),
            pltpu.VMEM((1, 4), jnp.float32),
        ],
    )(nt, req, upd, mean, std, emb, wa, wb, wc, r)


def _mid_epilogue(a_ref, t1a_ref, t1b_ref, t2b_ref, sv_ref, b_ref, r_ref):
    asum = a_ref[0] + a_ref[1]
    h = t1b_ref[:, :12]
    r = r_ref[...]
    wself = jnp.exp(
        _leaky(t1a_ref[:, 12:16] + t2b_ref[:, 12:16]) - sv_ref[0:1, 12:16])
    num = asum[:, :12] + h * jnp.dot(wself, r,
                                     preferred_element_type=jnp.float32)
    den4 = asum[:, 12:16] + wself
    den = jnp.dot(den4, r, preferred_element_type=jnp.float32)
    return num / (den + 1e-16) + b_ref[0:1, :12]


def _mid_body(a_ref, t1a_ref, t1b_ref, t2b_ref, sv_ref, b_ref, r_ref,
              wa_ref, wb_ref, wc_ref,
              t1ao_ref, t1bo_ref, t2bo_ref, svo_ref, ms_ref, md_ref,
              *, n, nblk):
    i = pl.program_id(0)
    out = _mid_epilogue(a_ref, t1a_ref, t1b_ref, t2b_ref, sv_ref, b_ref,
                        r_ref)
    x = jnp.maximum(out, 0.0)
    t1a = jnp.dot(x, wa_ref[...], preferred_element_type=jnp.float32)
    h = jnp.dot(x, wb_ref[...], preferred_element_type=jnp.float32)
    t2b = jnp.dot(x, wc_ref[...], preferred_element_type=jnp.float32)
    t1ao_ref[...] = t1a
    t1bo_ref[...] = jnp.concatenate(
        [h, jnp.ones((h.shape[0], 4), jnp.float32)], axis=1)
    t2bo_ref[...] = t2b
    rid = i * _RB + lax.broadcasted_iota(jnp.int32, (t1a.shape[0], 1), 0)
    valid = rid < n
    bms = jnp.max(jnp.where(valid, t1a[:, 12:16], _NEG), axis=0,
                  keepdims=True)
    bmd = jnp.max(jnp.where(valid, t2b[:, 12:16], _NEG), axis=0,
                  keepdims=True)

    @pl.when(i == 0)
    def _():
        ms_ref[...] = bms
        md_ref[...] = bmd

    @pl.when(i > 0)
    def _():
        ms_ref[...] = jnp.maximum(ms_ref[...], bms)
        md_ref[...] = jnp.maximum(md_ref[...], bmd)

    @pl.when(i == nblk - 1)
    def _():
        svo_ref[...] = _svb16(_leaky(ms_ref[...] + md_ref[...]), r_ref[...])


def _call_mid(a, t1a, t1b, t2b, sv, b16, r, wa, wb, wc, npad, n):
    nblk = npad // _RB
    mat = lambda i: (i, 0)
    fix = lambda i: (0, 0)
    return pl.pallas_call(
        functools.partial(_mid_body, n=n, nblk=nblk),
        grid=(nblk,),
        in_specs=[
            pl.BlockSpec((2, _RB, 16), lambda i: (0, i, 0)),
            pl.BlockSpec((_RB, 16), mat),
            pl.BlockSpec((_RB, 16), mat),
            pl.BlockSpec((_RB, 16), mat),
            pl.BlockSpec((1, 16), fix),
            pl.BlockSpec((1, 16), fix),
            pl.BlockSpec((4, 12), fix),
            pl.BlockSpec((12, 16), fix),
            pl.BlockSpec((12, 12), fix),
            pl.BlockSpec((12, 16), fix),
        ],
        out_specs=(
            pl.BlockSpec((_RB, 16), mat),
            pl.BlockSpec((_RB, 16), mat),
            pl.BlockSpec((_RB, 16), mat),
            pl.BlockSpec((1, 16), fix),
        ),
        out_shape=(
            jax.ShapeDtypeStruct((npad, 16), jnp.float32),
            jax.ShapeDtypeStruct((npad, 16), jnp.float32),
            jax.ShapeDtypeStruct((npad, 16), jnp.float32),
            jax.ShapeDtypeStruct((1, 16), jnp.float32),
        ),
        scratch_shapes=[
            pltpu.VMEM((1, 4), jnp.float32),
            pltpu.VMEM((1, 4), jnp.float32),
        ],
    )(a, t1a, t1b, t2b, sv, b16, r, wa, wb, wc)


def _last_body(a_ref, t1a_ref, t1b_ref, t2b_ref, sv_ref, b_ref, r_ref,
               hl_ref):
    hl_ref[...] = _mid_epilogue(a_ref, t1a_ref, t1b_ref, t2b_ref, sv_ref,
                                b_ref, r_ref)


def _call_last(a, t1a, t1b, t2b, sv, b16, r, npad):
    nblk = npad // _RB
    mat = lambda i: (i, 0)
    fix = lambda i: (0, 0)
    return pl.pallas_call(
        _last_body,
        grid=(nblk,),
        in_specs=[
            pl.BlockSpec((2, _RB, 16), lambda i: (0, i, 0)),
            pl.BlockSpec((_RB, 16), mat),
            pl.BlockSpec((_RB, 16), mat),
            pl.BlockSpec((_RB, 16), mat),
            pl.BlockSpec((1, 16), fix),
            pl.BlockSpec((1, 16), fix),
            pl.BlockSpec((4, 12), fix),
        ],
        out_specs=pl.BlockSpec((_RB, 12), mat),
        out_shape=jax.ShapeDtypeStruct((npad, 12), jnp.float32),
    )(a, t1a, t1b, t2b, sv, b16, r)


def _sample_update(i, zl, rl, rid, m_ref, s_ref, bv_ref, bi_ref):
    bmax_rl = jnp.max(rl)
    bmax_z = jnp.max(zl)
    bidx = jnp.min(jnp.where(zl == bmax_z, rid, jnp.int32(2**30)))

    @pl.when(i == 0)
    def _():
        m_ref[0, 0] = _NEG
        s_ref[0, 0] = 0.0
        bv_ref[0, 0] = _NEG
        bi_ref[0, 0] = 0

    m_old = m_ref[0, 0]
    m_new = jnp.maximum(m_old, bmax_rl)
    s_ref[0, 0] = s_ref[0, 0] * jnp.exp(m_old - m_new) + jnp.sum(
        jnp.exp(rl - m_new))
    m_ref[0, 0] = m_new
    upd = bmax_z > bv_ref[0, 0]
    bv_ref[0, 0] = jnp.where(upd, bmax_z, bv_ref[0, 0])
    bi_ref[0, 0] = jnp.where(upd, bidx, bi_ref[0, 0])


def _fc_body(h_ref, am_ref, g_ref, w1_ref, b1_ref, wa_ref, ba_ref, wb_ref,
             bb_ref, wc_ref, bc_ref, wo_ref, bo_ref,
             rl_ref, act_ref, lse_ref, m_ref, s_ref, bv_ref, bi_ref,
             *, n, nblk):
    i = pl.program_id(0)
    x = jnp.maximum(
        jnp.dot(h_ref[...], w1_ref[...], preferred_element_type=jnp.float32)
        + b1_ref[...], 0.0)
    for w, b in ((wa_ref, ba_ref), (wb_ref, bb_ref), (wc_ref, bc_ref)):
        x = jnp.maximum(
            jnp.dot(x, w[...], preferred_element_type=jnp.float32) + b[...],
            0.0)
    sc = jnp.dot(x, wo_ref[...], preferred_element_type=jnp.float32) \
        + bo_ref[...]
    am = am_ref[...]
    rid = i * _RB + lax.broadcasted_iota(jnp.int32, (sc.shape[0], 1), 0)
    rm_head = jnp.where(am == 0.0, -jnp.inf,
                        jnp.where(jnp.isneginf(am), 0.0, am))
    rm = jnp.where(rid < _L, rm_head, am)
    rl = jnp.where(rid < n, sc + rm, _NEG)
    rl_ref[...] = rl
    zl = jnp.where(rid < n, rl + g_ref[...], _NEG)
    _sample_update(i, zl, rl, rid, m_ref, s_ref, bv_ref, bi_ref)

    @pl.when(i == nblk - 1)
    def _():
        lse_ref[0, 0] = m_ref[0, 0] + jnp.log(s_ref[0, 0])
        act_ref[0, 0] = bi_ref[0, 0]


def _call_fc(hl1, am, g1, p, npad, n):
    nblk = npad // _RB
    mat = lambda i: (i, 0)
    fix = lambda i: (0, 0)
    return pl.pallas_call(
        functools.partial(_fc_body, n=n, nblk=nblk),
        grid=(nblk,),
        in_specs=[
            pl.BlockSpec((_RB, 12), mat),
            pl.BlockSpec((_RB, 1), mat),
            pl.BlockSpec((_RB, 1), mat),
            pl.BlockSpec((12, 128), fix),
            pl.BlockSpec((1, 128), fix),
            pl.BlockSpec((128, 128), fix),
            pl.BlockSpec((1, 128), fix),
            pl.BlockSpec((128, 128), fix),
            pl.BlockSpec((1, 128), fix),
            pl.BlockSpec((128, 128), fix),
            pl.BlockSpec((1, 128), fix),
            pl.BlockSpec((128, 1), fix),
            pl.BlockSpec((1, 1), fix),
        ],
        out_specs=(
            pl.BlockSpec((_RB, 1), mat),
            pl.BlockSpec((1, 1), fix, memory_space=pltpu.SMEM),
            pl.BlockSpec((1, 1), fix, memory_space=pltpu.SMEM),
        ),
        out_shape=(
            jax.ShapeDtypeStruct((npad, 1), jnp.float32),
            jax.ShapeDtypeStruct((1, 1), jnp.int32),
            jax.ShapeDtypeStruct((1, 1), jnp.float32),
        ),
        scratch_shapes=[
            pltpu.SMEM((1, 1), jnp.float32),
            pltpu.SMEM((1, 1), jnp.float32),
            pltpu.SMEM((1, 1), jnp.float32),
            pltpu.SMEM((1, 1), jnp.int32),
        ],
    )(hl1, am, g1, p['fc1_W'], p['fc1_b'][None, :],
      p['mlp'][0]['W'], p['mlp'][0]['b'][None, :],
      p['mlp'][1]['W'], p['mlp'][1]['b'][None, :],
      p['mlp'][2]['W'], p['mlp'][2]['b'][None, :],
      p['out_W'], p['out_b'][None, :])


def _k2_body(h_ref, am_ref, g_ref, p_ref, a1_ref,
             nl_ref, act_ref, lse_ref, m_ref, s_ref, bv_ref, bi_ref,
             *, n, nblk):
    i = pl.program_id(0)
    v = jnp.dot(h_ref[...], p_ref[...], preferred_element_type=jnp.float32)
    rid = i * _RB + lax.broadcasted_iota(jnp.int32, (v.shape[0], 1), 0)
    a1 = a1_ref[0, 0]
    mask2 = jnp.where(rid == a1, 0.0, am_ref[...])
    nl = jnp.where(rid < n, v + mask2, _NEG)
    nl_ref[...] = nl
    zl = jnp.where(rid < n, nl + g_ref[...], _NEG)
    _sample_update(i, zl, nl, rid, m_ref, s_ref, bv_ref, bi_ref)

    @pl.when(i == nblk - 1)
    def _():
        lse_ref[0, 0] = m_ref[0, 0] + jnp.log(s_ref[0, 0])
        act_ref[0, 0] = bi_ref[0, 0]


def _call_k2(hl1, am, g2, pvec, a1, npad, n):
    nblk = npad // _RB
    mat = lambda i: (i, 0)
    fix = lambda i: (0, 0)
    return pl.pallas_call(
        functools.partial(_k2_body, n=n, nblk=nblk),
        grid=(nblk,),
        in_specs=[
            pl.BlockSpec((_RB, 12), mat),
            pl.BlockSpec((_RB, 1), mat),
            pl.BlockSpec((_RB, 1), mat),
            pl.BlockSpec((12, 1), fix),
            pl.BlockSpec((1, 1), fix, memory_space=pltpu.SMEM),
        ],
        out_specs=(
            pl.BlockSpec((_RB, 1), mat),
            pl.BlockSpec((1, 1), fix, memory_space=pltpu.SMEM),
            pl.BlockSpec((1, 1), fix, memory_space=pltpu.SMEM),
        ),
        out_shape=(
            jax.ShapeDtypeStruct((npad, 1), jnp.float32),
            jax.ShapeDtypeStruct((1, 1), jnp.int32),
            jax.ShapeDtypeStruct((1, 1), jnp.float32),
        ),
        scratch_shapes=[
            pltpu.SMEM((1, 1), jnp.float32),
            pltpu.SMEM((1, 1), jnp.float32),
            pltpu.SMEM((1, 1), jnp.float32),
            pltpu.SMEM((1, 1), jnp.int32),
        ],
    )(hl1, am, g2, pvec, a1)


def _build_sc(npad, ep):
    pt = ep // (_NC * _NS)
    nb = pt // _EB
    rows_tile = npad // _NS
    zch = rows_tile // 16
    mesh = plsc.VectorSubcoreMesh(core_axis_name="c", subcore_axis_name="s",
                                  num_cores=_NC)

    @functools.partial(
        pl.kernel,
        out_type=jax.ShapeDtypeStruct((_NC, npad, 16), jnp.float32),
        mesh=mesh,
        compiler_params=pltpu.CompilerParams(use_tc_tiling_on_sc=False),
        scratch_types=[
            pltpu.VMEM_SHARED((npad, 16), jnp.float32),
            pltpu.VMEM((16,), jnp.float32),
            [pltpu.VMEM((_EB,), jnp.int32) for _ in range(2)],
            [pltpu.VMEM((_EB,), jnp.int32) for _ in range(2)],
            [pltpu.VMEM((_EB, 16), jnp.float32) for _ in range(2)],
            [pltpu.VMEM((_EB, 16), jnp.float32) for _ in range(2)],
            [pltpu.VMEM((_EB, 16), jnp.float32) for _ in range(2)],
            pltpu.VMEM((_EB, 16), jnp.float32),
            [pltpu.SemaphoreType.DMA for _ in range(2)],
            [pltpu.SemaphoreType.DMA for _ in range(2)],
            [pltpu.SemaphoreType.DMA for _ in range(2)],
        ],
    )
    def sc_pass(src_h, dst_h, t1a_h, t1b_h, t2b_h, sv_h, out_h,
                a_sh, svv, src_v, dst_v, ba_v, bb_v, bc_v, out_v,
                sem1, sem2, sem3):
        cid = lax.axis_index("c")
        sid = lax.axis_index("s")
        zero16 = jnp.zeros((16,), jnp.float32)
        base_rows = sid * rows_tile

        @pl.loop(0, _EB)
        def _(i):
            out_v[i, :] = zero16

        @pl.loop(0, 16)
        def _(k):
            pltpu.sync_copy(out_v.at[pl.ds(0, zch), :],
                            a_sh.at[pl.ds(base_rows + k * zch, zch), :])

        plsc.subcore_barrier()
        pltpu.sync_copy(sv_h, svv)
        svb = svv[...]
        tile_base = (cid * _NS + sid) * pt

        def issue(b, ph):
            base = tile_base + b * _EB
            pltpu.sync_copy(src_h.at[pl.ds(base, _EB)], src_v[ph])
            pltpu.sync_copy(dst_h.at[pl.ds(base, _EB)], dst_v[ph])
            d1 = pltpu.make_async_copy(t1a_h.at[src_v[ph]], ba_v[ph],
                                       sem1[ph])
            d2 = pltpu.make_async_copy(t1b_h.at[src_v[ph]], bb_v[ph],
                                       sem2[ph])
            d3 = pltpu.make_async_copy(t2b_h.at[dst_v[ph]], bc_v[ph],
                                       sem3[ph])
            d1.start()
            d2.start()
            d3.start()
            return d1, d2, d3

        issue(0, 0)
        issue(1, 1)

        @pl.loop(0, nb // 2)
        def _(i):
            b0 = 2 * i
            for ph in range(2):
                d1, d2, d3 = (
                    pltpu.make_async_copy(t1a_h.at[src_v[ph]], ba_v[ph],
                                          sem1[ph]),
                    pltpu.make_async_copy(t1b_h.at[src_v[ph]], bb_v[ph],
                                          sem2[ph]),
                    pltpu.make_async_copy(t2b_h.at[dst_v[ph]], bc_v[ph],
                                          sem3[ph]),
                )
                d1.wait()
                d2.wait()
                d3.wait()
                for j in range(_EB):
                    al = ba_v[ph][j, :] + bc_v[ph][j, :]
                    al = jnp.maximum(al, al * 0.2)
                    w = jnp.exp(al - svb)
                    out_v[j, :] = bb_v[ph][j, :] * w
                pltpu.sync_copy(out_v, a_sh.at[dst_v[ph]], add=True)
                issue(b0 + ph + 2, ph)

        for ph in range(2):
            pltpu.make_async_copy(t1a_h.at[src_v[ph]], ba_v[ph],
                                  sem1[ph]).wait()
            pltpu.make_async_copy(t1b_h.at[src_v[ph]], bb_v[ph],
                                  sem2[ph]).wait()
            pltpu.make_async_copy(t2b_h.at[dst_v[ph]], bc_v[ph],
                                  sem3[ph]).wait()

        plsc.subcore_barrier()

        @pl.loop(0, 16)
        def _(k):
            sl = pl.ds(base_rows + k * zch, zch)
            pltpu.sync_copy(a_sh.at[sl, :], out_v.at[pl.ds(0, zch), :])
            pltpu.sync_copy(out_v.at[pl.ds(0, zch), :], out_h.at[cid, sl, :])

    return sc_pass


def _fold_gat(gat, r, d_pad):
    w = gat['W'].astype(jnp.float32)
    d_in = w.shape[0]
    ssrc = r.T * gat['att_src'].reshape(_HID)[:, None]
    sdst = r.T * gat['att_dst'].reshape(_HID)[:, None]
    hm = jnp.asarray(_HEADMAP)
    wa = (w @ ssrc)[:, hm]
    wc = (w @ sdst)[:, hm]
    wb = w
    if d_pad > d_in:
        pad = ((0, d_pad - d_in), (0, 0))
        wa = jnp.pad(wa, pad)
        wb = jnp.pad(wb, pad)
        wc = jnp.pad(wc, pad)
    return wa, wb, wc


def kernel(update_step, requests, latency, active_mask, params, node_type,
           edge_index):
    del latency
    n = node_type.shape[0]
    e = edge_index.shape[1]
    npad = -(-n // _RB) * _RB
    epb = _NC * _NS * _EB
    ep = -(-e // epb) * epb
    f32 = jnp.float32

    def col(a, dtype=f32):
        return jnp.pad(a.astype(dtype), (0, npad - n))[:, None]

    nt = col(node_type, jnp.int32)
    req = col(requests)
    upd = col(update_step)
    am = col(active_mask)
    src = jnp.concatenate([edge_index[0].astype(jnp.int32),
                           jnp.full((ep - e + 2 * _EB,), n, jnp.int32)])
    dst = jnp.concatenate([edge_index[1].astype(jnp.int32),
                           jnp.full((ep - e + 2 * _EB,), n, jnp.int32)])

    r = jnp.asarray(_R_NP)
    gats = params['gats']
    wa0, wb0, wc0 = _fold_gat(gats[0], r, 8)
    t1a, t1b, t2b, sv = _call_pre0(nt, req, upd,
                                   params['type_emb'].astype(f32),
                                   wa0, wb0, wc0, r, npad, n)
    sc_pass = _build_sc(npad, ep)
    for l in range(4):
        b16 = jnp.pad(gats[l]['b'].astype(f32), (0, 4))[None, :]
        a = sc_pass(src, dst, t1a, t1b, t2b, sv.reshape(16))
        if l < 3:
            wa, wb, wc = _fold_gat(gats[l + 1], r, 12)
            t1a, t1b, t2b, sv = _call_mid(a, t1a, t1b, t2b, sv, b16, r,
                                          wa, wb, wc, npad, n)
        else:
            hl1 = _call_last(a, t1a, t1b, t2b, sv, b16, r, npad)

    k1, k2 = jax.random.split(jax.random.key(42))
    g1 = jnp.pad(jax.random.gumbel(k1, (n,), f32), (0, npad - n))[:, None]
    g2 = jnp.pad(jax.random.gumbel(k2, (n,), f32), (0, npad - n))[:, None]
    rl, a1, lse1 = _call_fc(hl1, am, g1, params, npad, n)
    a1s = a1[0, 0]
    row = lax.dynamic_index_in_dim(hl1, a1s, axis=0, keepdims=False)
    pvec = jnp.tanh(row @ params['proj_W'].astype(f32)
                    + params['proj_b'].astype(f32))
    nl, a2, lse2 = _call_k2(hl1, am, g2, pvec[:, None], a1, npad, n)

    rl1 = rl[:n, 0]
    nl1 = nl[:n, 0]
    logits = jnp.stack([rl1, nl1])
    actions = jnp.stack([a1s, a2[0, 0]])
    log_probs = jnp.stack([rl1[a1s] - lse1[0, 0], nl1[a2[0, 0]] - lse2[0, 0]])
    return logits, actions, log_probs
